# Initial kernel scaffold; baseline (speedup 1.0000x reference)
import jax
import jax.numpy as jnp
from jax.experimental import pallas as pl


def kernel(pos, normal, inv_feat, edge_index, edge_feat, node_bin, graph_feat, W_e, W_pos, W_norm, W_hn, W_graph, W_escore, Wg1, al1, ar1, res1, b1, Wg2, al2, ar2, res2, b2):
    raise NotImplementedError("write your pallas kernel here")


# speedup vs baseline: 1.1098x; 1.1098x over previous
import functools

import jax
import jax.numpy as jnp
from jax.experimental import pallas as pl

N = 10000
E = 320000
SBIN = 64
D_INV = 128
D_G = 64


def _inv_new_body(inv_ref, agg_ref, w1_ref, w2_ref, crow_ref, o_ref):
    acc = jnp.dot(inv_ref[...], w1_ref[...], preferred_element_type=jnp.float32)
    acc += jnp.dot(agg_ref[...], w2_ref[...], preferred_element_type=jnp.float32)
    o_ref[...] = jnp.maximum(acc + crow_ref[...], 0.0)


def _inv_new_pallas(inv_feat, agg, W1, W2, crow):
    blk = 2000
    grid = (N // blk,)
    return pl.pallas_call(
        _inv_new_body,
        grid=grid,
        in_specs=[
            pl.BlockSpec((blk, D_INV), lambda i: (i, 0)),
            pl.BlockSpec((blk, D_INV), lambda i: (i, 0)),
            pl.BlockSpec((D_INV, D_INV), lambda i: (0, 0)),
            pl.BlockSpec((D_INV, D_INV), lambda i: (0, 0)),
            pl.BlockSpec((1, D_INV), lambda i: (0, 0)),
        ],
        out_specs=pl.BlockSpec((blk, D_INV), lambda i: (i, 0)),
        out_shape=jax.ShapeDtypeStruct((N, D_INV), jnp.float32),
    )(inv_feat, agg, W1, W2, crow)


def _gat(h, W, al, ar, Wres, b, heads, dout):
    n = h.shape[0]
    feat = (h @ W).reshape(n, heads, dout)
    el = jnp.sum(feat * al[None, :, :], axis=-1)
    er = jnp.sum(feat * ar[None, :, :], axis=-1)
    e = jax.nn.leaky_relu(el[:, None, :] + er[None, :, :], negative_slope=0.2)
    alpha = jax.nn.softmax(e, axis=0)
    rst = jnp.einsum('sdh,sho->dho', alpha, feat)
    rst = rst + (h @ Wres).reshape(n, heads, dout)
    rst = rst + b.reshape(1, heads, dout)
    return jax.nn.elu(rst)


def kernel(pos, normal, inv_feat, edge_index, edge_feat, node_bin, graph_feat,
           W_e, W_pos, W_norm, W_hn, W_graph, W_escore,
           Wg1, al1, ar1, res1, b1, Wg2, al2, ar2, res2, b2):
    src = edge_index[0]
    dst = edge_index[1]

    W1 = W_e[:D_INV]
    W2 = W_e[D_INV:2 * D_INV]
    r_uv = W_e[2 * D_INV]
    r_cos = W_e[2 * D_INV + 1]
    W3 = W_e[2 * D_INV + 2:2 * D_INV + 2 + 16]
    W4 = W_e[2 * D_INV + 2 + 16:]
    c_row = graph_feat @ W4

    P1 = inv_feat @ W1
    P2 = inv_feat @ W2

    pos_s = pos[src]
    pos_d = pos[dst]
    nrm_s = normal[src]
    nrm_d = normal[dst]
    pos_sub = pos_d - pos_s
    normal_sub = nrm_d - nrm_s
    uv = jnp.sqrt(jnp.sum(pos_sub * pos_sub, axis=1))
    cos = jnp.sum(nrm_d * nrm_s, axis=1)

    geo = jnp.maximum(
        P1[src] + P2[dst] + edge_feat @ W3
        + uv[:, None] * r_uv[None, :] + cos[:, None] * r_cos[None, :]
        + c_row, 0.0)

    pw = jnp.maximum(geo @ W_pos, 0.0)
    nw = jnp.maximum(geo @ W_norm, 0.0)

    cnt = jax.ops.segment_sum(jnp.ones((E, 1), jnp.float32), dst, num_segments=N)
    inv_cnt = 1.0 / jnp.maximum(cnt, 1.0)
    pos_new = pos + jax.ops.segment_sum(pos_sub * pw, dst, num_segments=N) * inv_cnt
    nrm_new = normal + jax.ops.segment_sum(normal_sub * nw, dst, num_segments=N) * inv_cnt
    nrm_new = nrm_new / jnp.linalg.norm(nrm_new, axis=1, keepdims=True)
    agg = jax.ops.segment_sum(geo, dst, num_segments=N)

    Wh1 = W_hn[:D_INV]
    Wh2 = W_hn[D_INV:2 * D_INV]
    Wh3 = W_hn[2 * D_INV:]
    crow_hn = graph_feat @ Wh3
    inv_new = _inv_new_pallas(inv_feat, agg, Wh1, Wh2, crow_hn)

    score = jnp.ones((E, 1), jnp.float32)

    bs = jax.ops.segment_sum(inv_new, node_bin, num_segments=SBIN)
    bc = jax.ops.segment_sum(jnp.ones((N, 1), jnp.float32), node_bin, num_segments=SBIN)
    bin_mean = bs / jnp.maximum(bc, 1.0)
    h1 = _gat(bin_mean, Wg1, al1, ar1, res1, b1, 2, 128).reshape(SBIN, -1)
    h2 = _gat(h1, Wg2, al2, ar2, res2, b2, 2, 64).reshape(SBIN, -1)
    node_graph = jnp.mean(h2, axis=0, keepdims=True)
    edge_read = jnp.mean(geo, axis=0, keepdims=True)
    gfeat = jnp.maximum(
        jnp.concatenate([node_graph, edge_read, graph_feat], axis=1) @ W_graph, 0.0)
    return (pos_new, nrm_new, inv_new, geo, score, gfeat)


# speedup vs baseline: 2.8016x; 2.5244x over previous
import functools

import jax
import jax.numpy as jnp
from jax import lax
from jax.experimental import pallas as pl
from jax.experimental.pallas import tpu as pltpu
from jax.experimental.pallas import tpu_sc as plsc

N = 10000
E = 320000
SBIN = 64
D = 128

NC = 2
NS = 16
NW = NC * NS
EPT = E // NW
G = 50
CH = 200
NG = CH // G
NCH = EPT // CH
RPT = 624
RPT_LAST = N - 15 * RPT

_mesh = plsc.VectorSubcoreMesh(core_axis_name="c", subcore_axis_name="s")


def _prep_body(inv_ref, pos_ref, nrm_ref, w1_ref, w2_ref,
               ap_ref, bp_ref, at_ref, bt_ref):
    inv = inv_ref[...]
    ap_ref[...] = jnp.dot(inv, w1_ref[...], preferred_element_type=jnp.float32)
    bp_ref[...] = jnp.dot(inv, w2_ref[...], preferred_element_type=jnp.float32)
    pos = pos_ref[...]
    nrm = nrm_ref[...]
    n2 = jnp.sum(nrm * nrm, axis=1, keepdims=True)
    blk = pos.shape[0]
    z = jnp.zeros((blk, D - 7), jnp.float32)
    at_ref[...] = jnp.concatenate([-pos, -nrm, n2, z], axis=1)
    bt_ref[...] = jnp.concatenate([pos, nrm, n2, z], axis=1)


def _prep(inv_feat, pos, normal, W1, W2):
    blk = 2000
    return pl.pallas_call(
        _prep_body,
        grid=(N // blk,),
        in_specs=[
            pl.BlockSpec((blk, D), lambda i: (i, 0)),
            pl.BlockSpec((blk, 3), lambda i: (i, 0)),
            pl.BlockSpec((blk, 3), lambda i: (i, 0)),
            pl.BlockSpec((D, D), lambda i: (0, 0)),
            pl.BlockSpec((D, D), lambda i: (0, 0)),
        ],
        out_specs=[
            pl.BlockSpec((blk, D), lambda i: (i, 0)),
            pl.BlockSpec((blk, D), lambda i: (i, 0)),
            pl.BlockSpec((blk, D), lambda i: (i, 0)),
            pl.BlockSpec((blk, D), lambda i: (i, 0)),
        ],
        out_shape=[
            jax.ShapeDtypeStruct((N, D), jnp.float32),
            jax.ShapeDtypeStruct((N, D), jnp.float32),
            jax.ShapeDtypeStruct((N, D), jnp.float32),
            jax.ShapeDtypeStruct((N, D), jnp.float32),
        ],
    )(inv_feat, pos, normal, W1, W2)


@functools.partial(
    pl.kernel,
    mesh=_mesh,
    out_type=[
        jax.ShapeDtypeStruct((E, D), jnp.float32),
        jax.ShapeDtypeStruct((E, 16), jnp.float32),
    ],
    scratch_types=[
        pltpu.VMEM((NG, G), jnp.int32),
        pltpu.VMEM((NG, G), jnp.int32),
        pltpu.VMEM((CH, D), jnp.float32),
        pltpu.VMEM((CH, D), jnp.float32),
        pltpu.VMEM((CH, D), jnp.float32),
        pltpu.VMEM((CH, D), jnp.float32),
        pltpu.VMEM((CH, 16), jnp.float32),
        pltpu.SemaphoreType.DMA,
    ],
)
def _sc_gather(ap_hbm, bp_hbm, at_hbm, bt_hbm, src_hbm, dst_hbm,
               gp_hbm, gt_hbm, sidx, didx, bufa, bufb, buta, butb, gtbuf, sem):
    c = lax.axis_index("c")
    s = lax.axis_index("s")
    wid = s * NC + c
    ebase = wid * EPT

    def chunk(k, carry):
        pltpu.sync_copy(src_hbm.at[wid].at[k], sidx)
        pltpu.sync_copy(dst_hbm.at[wid].at[k], didx)
        descs = []
        for j in range(NG):
            sl = pl.ds(j * G, G)
            descs.append(pltpu.async_copy(ap_hbm.at[sidx.at[j]], bufa.at[sl], sem))
            descs.append(pltpu.async_copy(bp_hbm.at[didx.at[j]], bufb.at[sl], sem))
            descs.append(pltpu.async_copy(at_hbm.at[sidx.at[j]], buta.at[sl], sem))
            descs.append(pltpu.async_copy(bt_hbm.at[didx.at[j]], butb.at[sl], sem))
        for d_ in descs:
            d_.wait()

        def addp(i, _):
            r = i // 8
            q = (i % 8) * 16
            bufa[r, pl.ds(q, 16)] = bufa[r, pl.ds(q, 16)] + bufb[r, pl.ds(q, 16)]
            return 0

        def addt(r, _):
            gtbuf[r, :] = buta[r, pl.ds(0, 16)] + butb[r, pl.ds(0, 16)]
            return 0

        lax.fori_loop(0, CH * 8, addp, 0)
        lax.fori_loop(0, CH, addt, 0)
        erow = pl.ds(ebase + k * CH, CH)
        pltpu.sync_copy(bufa, gp_hbm.at[erow])
        pltpu.sync_copy(gtbuf, gt_hbm.at[erow])
        return carry

    lax.fori_loop(0, NCH, chunk, 0)


def _edge_body(gp_ref, gt_ref, ef_ref, w3_ref, ruv_ref, rcos_ref,
               crow_ref, wpos_ref, wnorm_ref, geo_ref, small_ref):
    gp = gp_ref[...]
    gt = gt_ref[...]
    psub = gt[:, 0:3]
    nsub = gt[:, 3:6]
    sn2 = gt[:, 6:7]
    uv = jnp.sqrt(jnp.sum(psub * psub, axis=1, keepdims=True))
    cosv = 0.5 * (sn2 - jnp.sum(nsub * nsub, axis=1, keepdims=True))
    geo = gp + jnp.dot(ef_ref[...], w3_ref[...],
                       preferred_element_type=jnp.float32)
    geo = geo + uv * ruv_ref[...] + cosv * rcos_ref[...] + crow_ref[...]
    geo = jnp.maximum(geo, 0.0)
    geo_ref[...] = geo
    pw = jnp.maximum(jnp.sum(geo * wpos_ref[...], axis=1, keepdims=True), 0.0)
    nw = jnp.maximum(jnp.sum(geo * wnorm_ref[...], axis=1, keepdims=True), 0.0)
    blk = gt.shape[0]
    one = jnp.ones((blk, 1), jnp.float32)
    small_ref[...] = jnp.concatenate(
        [psub * pw, nsub * nw, one, jnp.zeros((blk, 1), jnp.float32)], axis=1)


def _edge(gp, gt, ef, W3, r_uv, r_cos, crow, wposT, wnormT):
    blk = 1000
    return pl.pallas_call(
        _edge_body,
        grid=(E // blk,),
        in_specs=[
            pl.BlockSpec((blk, D), lambda i: (i, 0)),
            pl.BlockSpec((blk, 16), lambda i: (i, 0)),
            pl.BlockSpec((blk, 16), lambda i: (i, 0)),
            pl.BlockSpec((16, D), lambda i: (0, 0)),
            pl.BlockSpec((1, D), lambda i: (0, 0)),
            pl.BlockSpec((1, D), lambda i: (0, 0)),
            pl.BlockSpec((1, D), lambda i: (0, 0)),
            pl.BlockSpec((1, D), lambda i: (0, 0)),
            pl.BlockSpec((1, D), lambda i: (0, 0)),
        ],
        out_specs=[
            pl.BlockSpec((blk, D), lambda i: (i, 0)),
            pl.BlockSpec((blk, 8), lambda i: (i, 0)),
        ],
        out_shape=[
            jax.ShapeDtypeStruct((E, D), jnp.float32),
            jax.ShapeDtypeStruct((E, 8), jnp.float32),
        ],
    )(gp, gt, ef, W3, r_uv, r_cos, crow, wposT, wnormT)


CHS = 80
NCHS = EPT // CHS


@functools.partial(
    pl.kernel,
    mesh=_mesh,
    out_type=[
        jax.ShapeDtypeStruct((NC, N, D), jnp.float32),
        jax.ShapeDtypeStruct((NC, N, 8), jnp.float32),
    ],
    scratch_types=[
        pltpu.VMEM((1, CHS), jnp.int32),
        pltpu.VMEM((CHS, D), jnp.float32),
        pltpu.VMEM((CHS, 8), jnp.float32),
        pltpu.VMEM_SHARED((N, D), jnp.float32),
        pltpu.VMEM_SHARED((N, 8), jnp.float32),
    ],
)
def _sc_scatter(geo_hbm, small_hbm, dst4_hbm, z1_hbm, z2_hbm,
                agg_hbm, agg2_hbm, didx, geob, smallb, agg_sp, agg2_sp):
    c = lax.axis_index("c")
    s = lax.axis_index("s")
    wid = s * NC + c
    ebase = wid * EPT

    @pl.when(s < 15)
    def _():
        rows = pl.ds(s * RPT, RPT)
        pltpu.sync_copy(z1_hbm.at[pl.ds(0, RPT)], agg_sp.at[rows])
        pltpu.sync_copy(z2_hbm.at[pl.ds(0, RPT)], agg2_sp.at[rows])

    @pl.when(s == 15)
    def _():
        rows = pl.ds(15 * RPT, RPT_LAST)
        pltpu.sync_copy(z1_hbm, agg_sp.at[rows])
        pltpu.sync_copy(z2_hbm, agg2_sp.at[rows])

    plsc.subcore_barrier()

    def chunk(k, carry):
        pltpu.sync_copy(dst4_hbm.at[wid].at[k], didx)
        erow = pl.ds(ebase + k * CHS, CHS)
        pltpu.sync_copy(geo_hbm.at[erow], geob)
        pltpu.sync_copy(small_hbm.at[erow], smallb)
        pltpu.sync_copy(geob, agg_sp.at[didx.at[0]], add=True)
        pltpu.sync_copy(smallb, agg2_sp.at[didx.at[0]], add=True)
        return carry

    lax.fori_loop(0, NCHS, chunk, 0)
    plsc.subcore_barrier()

    @pl.when(s < 15)
    def _():
        rows = pl.ds(s * RPT, RPT)
        pltpu.sync_copy(agg_sp.at[rows], agg_hbm.at[c].at[rows])
        pltpu.sync_copy(agg2_sp.at[rows], agg2_hbm.at[c].at[rows])

    @pl.when(s == 15)
    def _():
        rows = pl.ds(15 * RPT, RPT_LAST)
        pltpu.sync_copy(agg_sp.at[rows], agg_hbm.at[c].at[rows])
        pltpu.sync_copy(agg2_sp.at[rows], agg2_hbm.at[c].at[rows])

    @pl.when(s < 15)
    def _():
        rows = pl.ds(s * RPT, RPT)
        pltpu.sync_copy(agg_sp.at[rows], agg_hbm.at[c].at[rows])

    @pl.when(s == 15)
    def _():
        rows = pl.ds(15 * RPT, RPT_LAST)
        pltpu.sync_copy(agg_sp.at[rows], agg_hbm.at[c].at[rows])


def _node_body(aggp_ref, inv_ref, w1_ref, w2_ref, crow_ref, o_ref):
    agg = aggp_ref[0] + aggp_ref[1]
    acc = jnp.dot(inv_ref[...], w1_ref[...], preferred_element_type=jnp.float32)
    acc += jnp.dot(agg, w2_ref[...], preferred_element_type=jnp.float32)
    o_ref[...] = jnp.maximum(acc + crow_ref[...], 0.0)


def _node(aggp, inv_feat, Wh1, Wh2, crow):
    blk = 2000
    return pl.pallas_call(
        _node_body,
        grid=(N // blk,),
        in_specs=[
            pl.BlockSpec((NC, blk, D), lambda i: (0, i, 0)),
            pl.BlockSpec((blk, D), lambda i: (i, 0)),
            pl.BlockSpec((D, D), lambda i: (0, 0)),
            pl.BlockSpec((D, D), lambda i: (0, 0)),
            pl.BlockSpec((1, D), lambda i: (0, 0)),
        ],
        out_specs=pl.BlockSpec((blk, D), lambda i: (i, 0)),
        out_shape=jax.ShapeDtypeStruct((N, D), jnp.float32),
    )(aggp, inv_feat, Wh1, Wh2, crow)


def _gat(h, W, al, ar, Wres, b, heads, dout):
    n = h.shape[0]
    feat = (h @ W).reshape(n, heads, dout)
    el = jnp.sum(feat * al[None, :, :], axis=-1)
    er = jnp.sum(feat * ar[None, :, :], axis=-1)
    e = jax.nn.leaky_relu(el[:, None, :] + er[None, :, :], negative_slope=0.2)
    alpha = jax.nn.softmax(e, axis=0)
    rst = jnp.einsum('sdh,sho->dho', alpha, feat)
    rst = rst + (h @ Wres).reshape(n, heads, dout)
    rst = rst + b.reshape(1, heads, dout)
    return jax.nn.elu(rst)


def kernel(pos, normal, inv_feat, edge_index, edge_feat, node_bin, graph_feat,
           W_e, W_pos, W_norm, W_hn, W_graph, W_escore,
           Wg1, al1, ar1, res1, b1, Wg2, al2, ar2, res2, b2):
    src4 = edge_index[0].reshape(NW, NCH, NG, G).astype(jnp.int32)
    dst4 = edge_index[1].reshape(NW, NCH, NG, G).astype(jnp.int32)
    dst4s = edge_index[1].reshape(NW, NCHS, 1, CHS).astype(jnp.int32)

    W1 = W_e[:D]
    W2 = W_e[D:2 * D]
    r_uv = W_e[2 * D:2 * D + 1]
    r_cos = W_e[2 * D + 1:2 * D + 2]
    W3 = W_e[2 * D + 2:2 * D + 2 + 16]
    W4 = W_e[2 * D + 2 + 16:]
    crow = graph_feat @ W4

    ap, bp, at, bt = _prep(inv_feat, pos, normal, W1, W2)
    gp, gt = _sc_gather(ap, bp, at, bt, src4, dst4)
    geo, small = _edge(gp, gt, edge_feat, W3, r_uv, r_cos, crow,
                       W_pos.reshape(1, D), W_norm.reshape(1, D))
    z1 = jnp.zeros((RPT_LAST, D), jnp.float32)
    z2 = jnp.zeros((RPT_LAST, 8), jnp.float32)
    _BISECT_SC_SCATTER = False
    if _BISECT_SC_SCATTER:
        aggp, smallp = _sc_scatter(geo, small, dst4s, z1, z2)
        S = smallp[0] + smallp[1]
    else:
        dstf = edge_index[1]
        agg1 = jax.ops.segment_sum(geo, dstf, num_segments=N)
        aggp = jnp.stack([agg1, jnp.zeros_like(agg1)])
        S = jax.ops.segment_sum(small, dstf, num_segments=N)
    cnt = S[:, 6:7]
    inv_cnt = 1.0 / jnp.maximum(cnt, 1.0)
    pos_new = pos + S[:, 0:3] * inv_cnt
    nrm_new = normal + S[:, 3:6] * inv_cnt
    nrm_new = nrm_new / jnp.linalg.norm(nrm_new, axis=1, keepdims=True)

    Wh1 = W_hn[:D]
    Wh2 = W_hn[D:2 * D]
    Wh3 = W_hn[2 * D:]
    inv_new = _node(aggp, inv_feat, Wh1, Wh2, graph_feat @ Wh3)

    score = jnp.ones((E, 1), jnp.float32)

    bs = jax.ops.segment_sum(inv_new, node_bin, num_segments=SBIN)
    bc = jax.ops.segment_sum(jnp.ones((N, 1), jnp.float32), node_bin,
                             num_segments=SBIN)
    bin_mean = bs / jnp.maximum(bc, 1.0)
    h1 = _gat(bin_mean, Wg1, al1, ar1, res1, b1, 2, 128).reshape(SBIN, -1)
    h2 = _gat(h1, Wg2, al2, ar2, res2, b2, 2, 64).reshape(SBIN, -1)
    node_graph = jnp.mean(h2, axis=0, keepdims=True)
    edge_read = jnp.sum(aggp[0] + aggp[1], axis=0, keepdims=True) / float(E)
    gfeat = jnp.maximum(
        jnp.concatenate([node_graph, edge_read, graph_feat], axis=1) @ W_graph,
        0.0)
    return (pos_new, nrm_new, inv_new, geo, score, gfeat)


# speedup vs baseline: 3.2353x; 1.1548x over previous
import functools

import jax
import jax.numpy as jnp
from jax import lax
from jax.experimental import pallas as pl
from jax.experimental.pallas import tpu as pltpu
from jax.experimental.pallas import tpu_sc as plsc

N = 10000
E = 320000
SBIN = 64
D = 128

NC = 2
NS = 16
NW = NC * NS
EPT = E // NW
G = 50
CH = 200
NG = CH // G
NCH = EPT // CH
RPT = 624
RPT_LAST = N - 15 * RPT

_mesh = plsc.VectorSubcoreMesh(core_axis_name="c", subcore_axis_name="s")


def _prep_body(inv_ref, pos_ref, nrm_ref, w1_ref, w2_ref,
               ap_ref, bp_ref, at_ref, bt_ref):
    inv = inv_ref[...]
    ap_ref[...] = jnp.dot(inv, w1_ref[...], preferred_element_type=jnp.float32)
    bp_ref[...] = jnp.dot(inv, w2_ref[...], preferred_element_type=jnp.float32)
    pos = pos_ref[...]
    nrm = nrm_ref[...]
    n2 = jnp.sum(nrm * nrm, axis=1, keepdims=True)
    blk = pos.shape[0]
    z = jnp.zeros((blk, D - 7), jnp.float32)
    at_ref[...] = jnp.concatenate([-pos, -nrm, n2, z], axis=1)
    bt_ref[...] = jnp.concatenate([pos, nrm, n2, z], axis=1)


def _prep(inv_feat, pos, normal, W1, W2):
    blk = 2000
    return pl.pallas_call(
        _prep_body,
        grid=(N // blk,),
        in_specs=[
            pl.BlockSpec((blk, D), lambda i: (i, 0)),
            pl.BlockSpec((blk, 3), lambda i: (i, 0)),
            pl.BlockSpec((blk, 3), lambda i: (i, 0)),
            pl.BlockSpec((D, D), lambda i: (0, 0)),
            pl.BlockSpec((D, D), lambda i: (0, 0)),
        ],
        out_specs=[
            pl.BlockSpec((blk, D), lambda i: (i, 0)),
            pl.BlockSpec((blk, D), lambda i: (i, 0)),
            pl.BlockSpec((blk, D), lambda i: (i, 0)),
            pl.BlockSpec((blk, D), lambda i: (i, 0)),
        ],
        out_shape=[
            jax.ShapeDtypeStruct((N, D), jnp.float32),
            jax.ShapeDtypeStruct((N, D), jnp.float32),
            jax.ShapeDtypeStruct((N, D), jnp.float32),
            jax.ShapeDtypeStruct((N, D), jnp.float32),
        ],
    )(inv_feat, pos, normal, W1, W2)


@functools.partial(
    pl.kernel,
    mesh=_mesh,
    out_type=[
        jax.ShapeDtypeStruct((E, D), jnp.float32),
        jax.ShapeDtypeStruct((E, 16), jnp.float32),
    ],
    scratch_types=[
        pltpu.VMEM((NG, G), jnp.int32),
        pltpu.VMEM((NG, G), jnp.int32),
        pltpu.VMEM((CH, D), jnp.float32),
        pltpu.VMEM((CH, D), jnp.float32),
        pltpu.VMEM((CH, D), jnp.float32),
        pltpu.VMEM((CH, D), jnp.float32),
        pltpu.VMEM((CH, 16), jnp.float32),
        pltpu.SemaphoreType.DMA,
    ],
)
def _sc_gather(ap_hbm, bp_hbm, at_hbm, bt_hbm, src_hbm, dst_hbm,
               gp_hbm, gt_hbm, sidx, didx, bufa, bufb, buta, butb, gtbuf, sem):
    c = lax.axis_index("c")
    s = lax.axis_index("s")
    wid = s * NC + c
    ebase = wid * EPT

    def chunk(k, carry):
        pltpu.sync_copy(src_hbm.at[wid].at[k], sidx)
        pltpu.sync_copy(dst_hbm.at[wid].at[k], didx)
        descs = []
        for j in range(NG):
            sl = pl.ds(j * G, G)
            descs.append(pltpu.async_copy(ap_hbm.at[sidx.at[j]], bufa.at[sl], sem))
            descs.append(pltpu.async_copy(bp_hbm.at[didx.at[j]], bufb.at[sl], sem))
            descs.append(pltpu.async_copy(at_hbm.at[sidx.at[j]], buta.at[sl], sem))
            descs.append(pltpu.async_copy(bt_hbm.at[didx.at[j]], butb.at[sl], sem))
        for d_ in descs:
            d_.wait()

        def addp(i, _):
            r = i // 8
            q = (i % 8) * 16
            bufa[r, pl.ds(q, 16)] = bufa[r, pl.ds(q, 16)] + bufb[r, pl.ds(q, 16)]
            return 0

        def addt(r, _):
            gtbuf[r, :] = buta[r, pl.ds(0, 16)] + butb[r, pl.ds(0, 16)]
            return 0

        lax.fori_loop(0, CH * 8, addp, 0)
        lax.fori_loop(0, CH, addt, 0)
        erow = pl.ds(ebase + k * CH, CH)
        pltpu.sync_copy(bufa, gp_hbm.at[erow])
        pltpu.sync_copy(gtbuf, gt_hbm.at[erow])
        return carry

    lax.fori_loop(0, NCH, chunk, 0)


def _edge_body(gp_ref, gt_ref, ef_ref, w3_ref, ruv_ref, rcos_ref,
               crow_ref, wpos_ref, wnorm_ref, geo_ref, small_ref):
    gp = gp_ref[...]
    gt = gt_ref[...]
    psub = gt[:, 0:3]
    nsub = gt[:, 3:6]
    sn2 = gt[:, 6:7]
    uv = jnp.sqrt(jnp.sum(psub * psub, axis=1, keepdims=True))
    cosv = 0.5 * (sn2 - jnp.sum(nsub * nsub, axis=1, keepdims=True))
    geo = gp + jnp.dot(ef_ref[...], w3_ref[...],
                       preferred_element_type=jnp.float32)
    geo = geo + uv * ruv_ref[...] + cosv * rcos_ref[...] + crow_ref[...]
    geo = jnp.maximum(geo, 0.0)
    geo_ref[...] = geo
    pw = jnp.maximum(jnp.sum(geo * wpos_ref[...], axis=1, keepdims=True), 0.0)
    nw = jnp.maximum(jnp.sum(geo * wnorm_ref[...], axis=1, keepdims=True), 0.0)
    blk = gt.shape[0]
    one = jnp.ones((blk, 1), jnp.float32)
    small_ref[...] = jnp.concatenate(
        [psub * pw, nsub * nw, one, jnp.zeros((blk, 1), jnp.float32)], axis=1)


def _edge(gp, gt, ef, W3, r_uv, r_cos, crow, wposT, wnormT):
    blk = 1000
    return pl.pallas_call(
        _edge_body,
        grid=(E // blk,),
        in_specs=[
            pl.BlockSpec((blk, D), lambda i: (i, 0)),
            pl.BlockSpec((blk, 16), lambda i: (i, 0)),
            pl.BlockSpec((blk, 16), lambda i: (i, 0)),
            pl.BlockSpec((16, D), lambda i: (0, 0)),
            pl.BlockSpec((1, D), lambda i: (0, 0)),
            pl.BlockSpec((1, D), lambda i: (0, 0)),
            pl.BlockSpec((1, D), lambda i: (0, 0)),
            pl.BlockSpec((1, D), lambda i: (0, 0)),
            pl.BlockSpec((1, D), lambda i: (0, 0)),
        ],
        out_specs=[
            pl.BlockSpec((blk, D), lambda i: (i, 0)),
            pl.BlockSpec((blk, 8), lambda i: (i, 0)),
        ],
        out_shape=[
            jax.ShapeDtypeStruct((E, D), jnp.float32),
            jax.ShapeDtypeStruct((E, 8), jnp.float32),
        ],
    )(gp, gt, ef, W3, r_uv, r_cos, crow, wposT, wnormT)


CHS = 80
NCHS = EPT // CHS


@functools.partial(
    pl.kernel,
    mesh=_mesh,
    out_type=[
        jax.ShapeDtypeStruct((NC, N, D), jnp.float32),
    ],
    scratch_types=[
        pltpu.VMEM((1, CHS), jnp.int32),
        pltpu.VMEM((CHS, D), jnp.float32),
        pltpu.VMEM_SHARED((N, D), jnp.float32),
    ],
)
def _sc_scatter(geo_hbm, dst4_hbm, z1_hbm, agg_hbm, didx, geob, agg_sp):
    c = lax.axis_index("c")
    s = lax.axis_index("s")
    wid = s * NC + c
    ebase = wid * EPT

    @pl.when(s < 15)
    def _():
        pltpu.sync_copy(z1_hbm.at[pl.ds(0, RPT)], agg_sp.at[pl.ds(s * RPT, RPT)])

    @pl.when(s == 15)
    def _():
        pltpu.sync_copy(z1_hbm, agg_sp.at[pl.ds(15 * RPT, RPT_LAST)])

    plsc.subcore_barrier()

    def chunk(k, carry):
        pltpu.sync_copy(dst4_hbm.at[wid].at[k], didx)
        erow = pl.ds(ebase + k * CHS, CHS)
        pltpu.sync_copy(geo_hbm.at[erow], geob)
        pltpu.sync_copy(geob, agg_sp.at[didx.at[0]], add=True)
        return carry

    lax.fori_loop(0, NCHS, chunk, 0)
    plsc.subcore_barrier()

    @pl.when(s < 15)
    def _():
        rows = pl.ds(s * RPT, RPT)
        pltpu.sync_copy(agg_sp.at[rows], agg_hbm.at[c].at[rows])

    @pl.when(s == 15)
    def _():
        rows = pl.ds(15 * RPT, RPT_LAST)
        pltpu.sync_copy(agg_sp.at[rows], agg_hbm.at[c].at[rows])

    @pl.when(s < 15)
    def _():
        rows = pl.ds(s * RPT, RPT)
        pltpu.sync_copy(agg_sp.at[rows], agg_hbm.at[c].at[rows])

    @pl.when(s == 15)
    def _():
        rows = pl.ds(15 * RPT, RPT_LAST)
        pltpu.sync_copy(agg_sp.at[rows], agg_hbm.at[c].at[rows])


def _node_body(aggp_ref, inv_ref, w1_ref, w2_ref, crow_ref, o_ref):
    agg = aggp_ref[0] + aggp_ref[1]
    acc = jnp.dot(inv_ref[...], w1_ref[...], preferred_element_type=jnp.float32)
    acc += jnp.dot(agg, w2_ref[...], preferred_element_type=jnp.float32)
    o_ref[...] = jnp.maximum(acc + crow_ref[...], 0.0)


def _node(aggp, inv_feat, Wh1, Wh2, crow):
    blk = 2000
    return pl.pallas_call(
        _node_body,
        grid=(N // blk,),
        in_specs=[
            pl.BlockSpec((NC, blk, D), lambda i: (0, i, 0)),
            pl.BlockSpec((blk, D), lambda i: (i, 0)),
            pl.BlockSpec((D, D), lambda i: (0, 0)),
            pl.BlockSpec((D, D), lambda i: (0, 0)),
            pl.BlockSpec((1, D), lambda i: (0, 0)),
        ],
        out_specs=pl.BlockSpec((blk, D), lambda i: (i, 0)),
        out_shape=jax.ShapeDtypeStruct((N, D), jnp.float32),
    )(aggp, inv_feat, Wh1, Wh2, crow)


def _gat(h, W, al, ar, Wres, b, heads, dout):
    n = h.shape[0]
    feat = (h @ W).reshape(n, heads, dout)
    el = jnp.sum(feat * al[None, :, :], axis=-1)
    er = jnp.sum(feat * ar[None, :, :], axis=-1)
    e = jax.nn.leaky_relu(el[:, None, :] + er[None, :, :], negative_slope=0.2)
    alpha = jax.nn.softmax(e, axis=0)
    rst = jnp.einsum('sdh,sho->dho', alpha, feat)
    rst = rst + (h @ Wres).reshape(n, heads, dout)
    rst = rst + b.reshape(1, heads, dout)
    return jax.nn.elu(rst)


def kernel(pos, normal, inv_feat, edge_index, edge_feat, node_bin, graph_feat,
           W_e, W_pos, W_norm, W_hn, W_graph, W_escore,
           Wg1, al1, ar1, res1, b1, Wg2, al2, ar2, res2, b2):
    src4 = edge_index[0].reshape(NW, NCH, NG, G).astype(jnp.int32)
    dst4 = edge_index[1].reshape(NW, NCH, NG, G).astype(jnp.int32)
    dst4s = edge_index[1].reshape(NW, NCHS, 1, CHS).astype(jnp.int32)

    W1 = W_e[:D]
    W2 = W_e[D:2 * D]
    r_uv = W_e[2 * D:2 * D + 1]
    r_cos = W_e[2 * D + 1:2 * D + 2]
    W3 = W_e[2 * D + 2:2 * D + 2 + 16]
    W4 = W_e[2 * D + 2 + 16:]
    crow = graph_feat @ W4

    ap, bp, at, bt = _prep(inv_feat, pos, normal, W1, W2)
    gp, gt = _sc_gather(ap, bp, at, bt, src4, dst4)
    geo, small = _edge(gp, gt, edge_feat, W3, r_uv, r_cos, crow,
                       W_pos.reshape(1, D), W_norm.reshape(1, D))
    z1 = jnp.zeros((RPT_LAST, D), jnp.float32)
    z2 = jnp.zeros((RPT_LAST, 8), jnp.float32)
    (aggp,) = _sc_scatter(geo, dst4s, z1)
    S = jax.ops.segment_sum(small, edge_index[1], num_segments=N)
    cnt = S[:, 6:7]
    inv_cnt = 1.0 / jnp.maximum(cnt, 1.0)
    pos_new = pos + S[:, 0:3] * inv_cnt
    nrm_new = normal + S[:, 3:6] * inv_cnt
    nrm_new = nrm_new / jnp.linalg.norm(nrm_new, axis=1, keepdims=True)

    Wh1 = W_hn[:D]
    Wh2 = W_hn[D:2 * D]
    Wh3 = W_hn[2 * D:]
    inv_new = _node(aggp, inv_feat, Wh1, Wh2, graph_feat @ Wh3)

    score = jnp.ones((E, 1), jnp.float32)

    bs = jax.ops.segment_sum(inv_new, node_bin, num_segments=SBIN)
    bc = jax.ops.segment_sum(jnp.ones((N, 1), jnp.float32), node_bin,
                             num_segments=SBIN)
    bin_mean = bs / jnp.maximum(bc, 1.0)
    h1 = _gat(bin_mean, Wg1, al1, ar1, res1, b1, 2, 128).reshape(SBIN, -1)
    h2 = _gat(h1, Wg2, al2, ar2, res2, b2, 2, 64).reshape(SBIN, -1)
    node_graph = jnp.mean(h2, axis=0, keepdims=True)
    edge_read = jnp.sum(aggp[0] + aggp[1], axis=0, keepdims=True) / float(E)
    gfeat = jnp.maximum(
        jnp.concatenate([node_graph, edge_read, graph_feat], axis=1) @ W_graph,
        0.0)
    return (pos_new, nrm_new, inv_new, geo, score, gfeat)
